# baseline (speedup 1.0000x reference)
import functools

import jax
import jax.numpy as jnp
from jax import lax
from jax.experimental import pallas as pl
from jax.experimental.pallas import tpu as pltpu
from jax.experimental.pallas import tpu_sc as plsc

NC = 2
NS = 16
NW = NC * NS
L = 16
M = 65536
D = 128
RPW = M // NW
CH = 256
NCH = RPW // CH
IMAX = 2147483647

_mesh = plsc.VectorSubcoreMesh(core_axis_name="c", subcore_axis_name="s")
_params = pltpu.CompilerParams(needs_layout_passes=False)


def _scan_body(q_hbm, keys_hbm, mask_hbm, outv_hbm, outi_hbm,
               q_v, mask_v, kb0, kb1, tsp, sv, si, sem0, sem1):
    cid = lax.axis_index("c")
    sid = lax.axis_index("s")
    wid = sid * NC + cid
    base = wid * RPW

    pltpu.sync_copy(q_hbm, q_v)
    pltpu.sync_copy(mask_hbm.at[pl.ds(base, RPW)], mask_v)
    pltpu.async_copy(keys_hbm.at[pl.ds(base * D, CH * D)], kb0, sem0)
    pltpu.async_copy(keys_hbm.at[pl.ds((base + CH) * D, CH * D)], kb1, sem1)

    qv = [q_v[pl.ds(16 * j, 16)] for j in range(8)]
    iota = lax.iota(jnp.int32, L)
    neg = jnp.full((L,), -jnp.inf, jnp.float32)
    rm0 = neg
    ri0 = jnp.full((L,), IMAX, jnp.int32)

    def chunk_pair(i, carry):
        rm, ri = carry
        for b in range(2):
            k = 2 * i + b
            kb = (kb0, kb1)[b]
            sem = (sem0, sem1)[b]
            pltpu.make_async_copy(
                keys_hbm.at[pl.ds((base + k * CH) * D, CH * D)], kb, sem
            ).wait()

            def group(g, cr, kb=kb, k=k):
                rm2, ri2 = cr
                for r in range(L):
                    ro = (g * L + r) * D
                    a = kb[pl.ds(ro, 16)] * qv[0]
                    for j in range(1, 8):
                        a = a + kb[pl.ds(ro + 16 * j, 16)] * qv[j]
                    tsp[pl.ds(16 * r, 16)] = a
                ssum = plsc.load_gather(tsp, [iota * 16])
                for col in range(1, L):
                    ssum = ssum + plsc.load_gather(tsp, [iota * 16 + col])
                moff = k * CH + g * L
                mk = mask_v[pl.ds(moff, 16)]
                ssum = jnp.where(mk > 0.0, ssum, neg)
                gidx = iota + (base + moff)
                upd = ssum > rm2
                return jnp.where(upd, ssum, rm2), jnp.where(upd, gidx, ri2)

            rm, ri = lax.fori_loop(0, CH // L, group, (rm, ri))

            nk = k + 2

            @pl.when(nk < NCH)
            def _():
                pltpu.async_copy(
                    keys_hbm.at[pl.ds((base + nk * CH) * D, CH * D)], kb, sem
                )
        return rm, ri

    rm, ri = lax.fori_loop(0, NCH // 2, chunk_pair, (rm0, ri0))

    m = jnp.max(rm)
    cand = jnp.where(rm == m, ri, IMAX)
    it = jnp.min(cand)
    sv[...] = jnp.broadcast_to(m, (L,))
    si[...] = jnp.broadcast_to(it, (L,))
    pltpu.sync_copy(sv, outv_hbm.at[pl.ds(wid * L, L)])
    pltpu.sync_copy(si, outi_hbm.at[pl.ds(wid * L, L)])


_scan = functools.partial(
    pl.kernel,
    out_type=(
        jax.ShapeDtypeStruct((NW * L,), jnp.float32),
        jax.ShapeDtypeStruct((NW * L,), jnp.int32),
    ),
    mesh=_mesh,
    compiler_params=_params,
    scratch_types=[
        pltpu.VMEM((D,), jnp.float32),
        pltpu.VMEM((RPW,), jnp.float32),
        pltpu.VMEM((CH * D,), jnp.float32),
        pltpu.VMEM((CH * D,), jnp.float32),
        pltpu.VMEM((L * L,), jnp.float32),
        pltpu.VMEM((L,), jnp.float32),
        pltpu.VMEM((L,), jnp.int32),
        pltpu.SemaphoreType.DMA,
        pltpu.SemaphoreType.DMA,
    ],
)(_scan_body)


def _finalize_body(cv_hbm, ci_hbm, uc_hbm, pc_hbm, av_hbm,
                   ucout_hbm, act_hbm, outf_hbm, outi_hbm,
                   cvv, civ, ucb, pcv, idxv, abuf, stf, sti, sem, semu):
    cid = lax.axis_index("c")
    sid = lax.axis_index("s")
    wid = sid * NC + cid
    base = wid * RPW

    pltpu.async_copy(uc_hbm.at[pl.ds(base, RPW)], ucb, semu)
    pltpu.sync_copy(cv_hbm, cvv)
    pltpu.sync_copy(ci_hbm, civ)
    pltpu.sync_copy(pc_hbm, pcv)

    iota = lax.iota(jnp.int32, L)
    g0 = plsc.load_gather(cvv, [iota * L])
    g1 = plsc.load_gather(cvv, [iota * L + NW * L // 2])
    h0 = plsc.load_gather(civ, [iota * L])
    h1 = plsc.load_gather(civ, [iota * L + NW * L // 2])
    m = jnp.max(jnp.maximum(g0, g1))
    c0 = jnp.where(g0 == m, h0, IMAX)
    c1 = jnp.where(g1 == m, h1, IMAX)
    it = jnp.minimum(jnp.min(c0), jnp.min(c1))
    it = jnp.where(it == IMAX, 0, it)
    hit = m > 0.95

    pltpu.make_async_copy(uc_hbm.at[pl.ds(base, RPW)], ucb, semu).wait()
    off = jnp.clip(it - base, 0, RPW - 1)
    own = (iota == 0) & jnp.broadcast_to(
        hit & (it >= base) & (it < base + RPW), (L,))
    plsc.addupdate_scatter(
        ucb, [jnp.broadcast_to(off, (L,))], jnp.ones((L,), jnp.int32), mask=own)
    pltpu.sync_copy(ucb, ucout_hbm.at[pl.ds(base, RPW)])

    @pl.when(wid == 0)
    def _():
        itv = jnp.broadcast_to(it, (L,))
        idxv[...] = itv
        pltpu.async_copy(av_hbm.at[idxv], abuf, sem).wait()
        pltpu.sync_copy(abuf.at[0], act_hbm)
        stf[...] = jnp.broadcast_to(m, (L,))
        pltpu.sync_copy(stf, outf_hbm)
        newpc = jnp.where(jnp.broadcast_to(hit, (L,)), itv, pcv[...])
        sti[...] = jnp.where(iota == 0, itv, jnp.where(iota == 1, newpc, 0))
        pltpu.sync_copy(sti, outi_hbm)


_finalize = functools.partial(
    pl.kernel,
    out_type=(
        jax.ShapeDtypeStruct((M,), jnp.int32),
        jax.ShapeDtypeStruct((D,), jnp.float32),
        jax.ShapeDtypeStruct((L,), jnp.float32),
        jax.ShapeDtypeStruct((L,), jnp.int32),
    ),
    mesh=_mesh,
    compiler_params=_params,
    scratch_types=[
        pltpu.VMEM((NW * L,), jnp.float32),
        pltpu.VMEM((NW * L,), jnp.int32),
        pltpu.VMEM((RPW,), jnp.int32),
        pltpu.VMEM((L,), jnp.int32),
        pltpu.VMEM((L,), jnp.int32),
        pltpu.VMEM((L, D), jnp.float32),
        pltpu.VMEM((L,), jnp.float32),
        pltpu.VMEM((L,), jnp.int32),
        pltpu.SemaphoreType.DMA,
        pltpu.SemaphoreType.DMA,
    ],
)(_finalize_body)


def kernel(sensor_spikes, sensor_keys, action_values, is_active,
           usage_counts, program_counter):
    q = sensor_spikes.reshape(D)
    keys_flat = sensor_keys.reshape(M * D)
    maskf = is_active.astype(jnp.float32)
    pc16 = jnp.broadcast_to(program_counter, (L,)).astype(jnp.int32)

    cv, ci = _scan(q, keys_flat, maskf)
    ucn, act, outf, outi = _finalize(cv, ci, usage_counts, pc16, action_values)

    action = act.reshape(1, D)
    confidence = outf[0:1]
    best_idx = outi[0:1]
    new_pc = outi[1]
    return action, confidence, best_idx, ucn, new_pc


# speedup vs baseline: 1.2802x; 1.2802x over previous
import functools

import jax
import jax.numpy as jnp
from jax import lax
from jax.experimental import pallas as pl
from jax.experimental.pallas import tpu as pltpu
from jax.experimental.pallas import tpu_sc as plsc

NC = 2
NS = 16
NW = NC * NS
L = 16
M = 65536
D = 128
RPW = M // NW
CH = 256
NCH = RPW // CH
IMAX = 2147483647

_mesh = plsc.VectorSubcoreMesh(core_axis_name="c", subcore_axis_name="s")
_params = pltpu.CompilerParams(needs_layout_passes=False)


def _scan_body(q_hbm, keys_hbm, mask_hbm, outv_hbm, outi_hbm,
               q_v, mask_v, kb0, kb1, tsp, sv, si, sem0, sem1):
    cid = lax.axis_index("c")
    sid = lax.axis_index("s")
    wid = sid * NC + cid
    base = wid * RPW

    pltpu.sync_copy(q_hbm, q_v)
    pltpu.sync_copy(mask_hbm.at[pl.ds(base, RPW)], mask_v)
    pltpu.async_copy(keys_hbm.at[pl.ds(base * D, CH * D)], kb0, sem0)
    pltpu.async_copy(keys_hbm.at[pl.ds((base + CH) * D, CH * D)], kb1, sem1)

    qv = [q_v[pl.ds(16 * j, 16)] for j in range(8)]
    iota = lax.iota(jnp.int32, L)
    neg = jnp.full((L,), -jnp.inf, jnp.float32)
    rm0 = neg
    ri0 = jnp.full((L,), IMAX, jnp.int32)

    def chunk_pair(i, carry):
        rm, ri = carry
        for b in range(2):
            k = 2 * i + b
            kb = (kb0, kb1)[b]
            sem = (sem0, sem1)[b]
            pltpu.make_async_copy(
                keys_hbm.at[pl.ds((base + k * CH) * D, CH * D)], kb, sem
            ).wait()

            def group(g, cr, kb=kb, k=k):
                rm2, ri2 = cr
                tbase = g * (L * L)
                for r in range(L):
                    ro = (g * L + r) * D
                    a = kb[pl.ds(ro, 16)] * qv[0]
                    a2 = kb[pl.ds(ro + 16, 16)] * qv[1]
                    for j in range(2, 8, 2):
                        a = a + kb[pl.ds(ro + 16 * j, 16)] * qv[j]
                        a2 = a2 + kb[pl.ds(ro + 16 * (j + 1), 16)] * qv[j + 1]
                    tsp[pl.ds(tbase + L * r, 16)] = a + a2
                idxg = iota * L + tbase
                ssum = plsc.load_gather(tsp, [idxg])
                s2 = plsc.load_gather(tsp, [idxg + 1])
                for col in range(2, L, 2):
                    ssum = ssum + plsc.load_gather(tsp, [idxg + col])
                    s2 = s2 + plsc.load_gather(tsp, [idxg + col + 1])
                ssum = ssum + s2
                moff = k * CH + g * L
                mk = mask_v[pl.ds(moff, 16)]
                ssum = jnp.where(mk > 0.0, ssum, neg)
                gidx = iota + (base + moff)
                upd = ssum > rm2
                return jnp.where(upd, ssum, rm2), jnp.where(upd, gidx, ri2)

            rm, ri = plsc.parallel_loop(0, CH // L, carry=(rm, ri))(group)

            nk = k + 2

            @pl.when(nk < NCH)
            def _():
                pltpu.async_copy(
                    keys_hbm.at[pl.ds((base + nk * CH) * D, CH * D)], kb, sem
                )
        return rm, ri

    rm, ri = lax.fori_loop(0, NCH // 2, chunk_pair, (rm0, ri0))

    m = jnp.max(rm)
    cand = jnp.where(rm == m, ri, IMAX)
    it = jnp.min(cand)
    sv[...] = jnp.broadcast_to(m, (L,))
    si[...] = jnp.broadcast_to(it, (L,))
    pltpu.sync_copy(sv, outv_hbm.at[pl.ds(wid * L, L)])
    pltpu.sync_copy(si, outi_hbm.at[pl.ds(wid * L, L)])


_scan = functools.partial(
    pl.kernel,
    out_type=(
        jax.ShapeDtypeStruct((NW * L,), jnp.float32),
        jax.ShapeDtypeStruct((NW * L,), jnp.int32),
    ),
    mesh=_mesh,
    compiler_params=_params,
    scratch_types=[
        pltpu.VMEM((D,), jnp.float32),
        pltpu.VMEM((RPW,), jnp.float32),
        pltpu.VMEM((CH * D,), jnp.float32),
        pltpu.VMEM((CH * D,), jnp.float32),
        pltpu.VMEM(((CH // L) * L * L,), jnp.float32),
        pltpu.VMEM((L,), jnp.float32),
        pltpu.VMEM((L,), jnp.int32),
        pltpu.SemaphoreType.DMA,
        pltpu.SemaphoreType.DMA,
    ],
)(_scan_body)


def _finalize_body(cv_hbm, ci_hbm, uc_hbm, pc_hbm, av_hbm,
                   ucout_hbm, act_hbm, outf_hbm, outi_hbm,
                   cvv, civ, ucb, pcv, idxv, abuf, stf, sti, sem, semu):
    cid = lax.axis_index("c")
    sid = lax.axis_index("s")
    wid = sid * NC + cid
    base = wid * RPW

    pltpu.async_copy(uc_hbm.at[pl.ds(base, RPW)], ucb, semu)
    pltpu.sync_copy(cv_hbm, cvv)
    pltpu.sync_copy(ci_hbm, civ)
    pltpu.sync_copy(pc_hbm, pcv)

    iota = lax.iota(jnp.int32, L)
    g0 = plsc.load_gather(cvv, [iota * L])
    g1 = plsc.load_gather(cvv, [iota * L + NW * L // 2])
    h0 = plsc.load_gather(civ, [iota * L])
    h1 = plsc.load_gather(civ, [iota * L + NW * L // 2])
    m = jnp.max(jnp.maximum(g0, g1))
    c0 = jnp.where(g0 == m, h0, IMAX)
    c1 = jnp.where(g1 == m, h1, IMAX)
    it = jnp.minimum(jnp.min(c0), jnp.min(c1))
    it = jnp.where(it == IMAX, 0, it)
    hit = m > 0.95

    pltpu.make_async_copy(uc_hbm.at[pl.ds(base, RPW)], ucb, semu).wait()
    off = jnp.clip(it - base, 0, RPW - 1)
    own = (iota == 0) & jnp.broadcast_to(
        hit & (it >= base) & (it < base + RPW), (L,))
    plsc.addupdate_scatter(
        ucb, [jnp.broadcast_to(off, (L,))], jnp.ones((L,), jnp.int32), mask=own)
    pltpu.sync_copy(ucb, ucout_hbm.at[pl.ds(base, RPW)])

    @pl.when(wid == 0)
    def _():
        itv = jnp.broadcast_to(it, (L,))
        idxv[...] = itv
        pltpu.async_copy(av_hbm.at[idxv], abuf, sem).wait()
        pltpu.sync_copy(abuf.at[0], act_hbm)
        stf[...] = jnp.broadcast_to(m, (L,))
        pltpu.sync_copy(stf, outf_hbm)
        newpc = jnp.where(jnp.broadcast_to(hit, (L,)), itv, pcv[...])
        sti[...] = jnp.where(iota == 0, itv, jnp.where(iota == 1, newpc, 0))
        pltpu.sync_copy(sti, outi_hbm)


_finalize = functools.partial(
    pl.kernel,
    out_type=(
        jax.ShapeDtypeStruct((M,), jnp.int32),
        jax.ShapeDtypeStruct((D,), jnp.float32),
        jax.ShapeDtypeStruct((L,), jnp.float32),
        jax.ShapeDtypeStruct((L,), jnp.int32),
    ),
    mesh=_mesh,
    compiler_params=_params,
    scratch_types=[
        pltpu.VMEM((NW * L,), jnp.float32),
        pltpu.VMEM((NW * L,), jnp.int32),
        pltpu.VMEM((RPW,), jnp.int32),
        pltpu.VMEM((L,), jnp.int32),
        pltpu.VMEM((L,), jnp.int32),
        pltpu.VMEM((L, D), jnp.float32),
        pltpu.VMEM((L,), jnp.float32),
        pltpu.VMEM((L,), jnp.int32),
        pltpu.SemaphoreType.DMA,
        pltpu.SemaphoreType.DMA,
    ],
)(_finalize_body)


def kernel(sensor_spikes, sensor_keys, action_values, is_active,
           usage_counts, program_counter):
    q = sensor_spikes.reshape(D)
    keys_flat = sensor_keys.reshape(M * D)
    maskf = is_active.astype(jnp.float32)
    pc16 = jnp.broadcast_to(program_counter, (L,)).astype(jnp.int32)

    cv, ci = _scan(q, keys_flat, maskf)
    ucn, act, outf, outi = _finalize(cv, ci, usage_counts, pc16, action_values)

    action = act.reshape(1, D)
    confidence = outf[0:1]
    best_idx = outi[0:1]
    new_pc = outi[1]
    return action, confidence, best_idx, ucn, new_pc
